# Initial kernel scaffold; baseline (speedup 1.0000x reference)
import jax
import jax.numpy as jnp
from jax.experimental import pallas as pl


def kernel(table, d_seq):
    raise NotImplementedError("write your pallas kernel here")


# speedup vs baseline: 1.3423x; 1.3423x over previous
import functools

import jax
import jax.numpy as jnp
from jax import lax
from jax.experimental import pallas as pl
from jax.experimental.pallas import tpu as pltpu
from jax.experimental.pallas import tpu_sc as plsc

NC = 2
NS = 16
NW = NC * NS


def _sc_gather(table, idx, n, d):
    b_per_w = n // NW
    chunk = 32
    n_chunks = b_per_w // chunk

    mesh = plsc.VectorSubcoreMesh(core_axis_name="c", subcore_axis_name="s")

    @functools.partial(
        pl.kernel,
        out_type=jax.ShapeDtypeStruct((n, d), jnp.float32),
        mesh=mesh,
        scratch_types=[
            pltpu.VMEM((chunk,), jnp.int32),
            pltpu.VMEM((chunk, d), jnp.float32),
            pltpu.SemaphoreType.DMA,
        ],
    )
    def body(table_hbm, idx_hbm, out_hbm, idx_v, rows_v, sem):
        wid = lax.axis_index("s") * NC + lax.axis_index("c")
        base = wid * b_per_w

        def chunk_body(j, carry):
            rb = base + j * chunk
            pltpu.sync_copy(idx_hbm.at[pl.ds(rb, chunk)], idx_v)
            pltpu.async_copy(table_hbm.at[idx_v], rows_v, sem).wait()
            pltpu.sync_copy(rows_v, out_hbm.at[pl.ds(rb, chunk)])
            return carry

        lax.fori_loop(0, n_chunks, chunk_body, 0)

    return body(table, idx)


def kernel(table, d_seq):
    n = table.shape[0] - 1
    d = table.shape[1]
    idx = jnp.minimum(jnp.arange(n, dtype=jnp.int32),
                      jnp.asarray(d_seq, jnp.int32) - 1)
    return _sc_gather(table, idx, n, d)


# speedup vs baseline: 1.5485x; 1.1537x over previous
import functools

import jax
import jax.numpy as jnp
from jax import lax
from jax.experimental import pallas as pl
from jax.experimental.pallas import tpu as pltpu
from jax.experimental.pallas import tpu_sc as plsc

NC = 2
NS = 16
NW = NC * NS


def _sc_gather(table, idx, n, d):
    b_per_w = n // NW
    chunk = 32
    n_chunks = b_per_w // chunk

    mesh = plsc.VectorSubcoreMesh(core_axis_name="c", subcore_axis_name="s")

    @functools.partial(
        pl.kernel,
        out_type=jax.ShapeDtypeStruct((n, d), jnp.float32),
        mesh=mesh,
        scratch_types=[
            pltpu.VMEM((b_per_w,), jnp.int32),
            pltpu.VMEM((chunk, d), jnp.float32),
            pltpu.VMEM((chunk, d), jnp.float32),
            pltpu.SemaphoreType.DMA,
            pltpu.SemaphoreType.DMA,
            pltpu.SemaphoreType.DMA,
            pltpu.SemaphoreType.DMA,
        ],
    )
    def body(table_hbm, idx_hbm, out_hbm, idx_v, buf0, buf1, sg0, sg1,
             sw0, sw1):
        wid = lax.axis_index("s") * NC + lax.axis_index("c")
        base = wid * b_per_w
        bufs, sgs, sws = (buf0, buf1), (sg0, sg1), (sw0, sw1)

        pltpu.sync_copy(idx_hbm.at[pl.ds(base, b_per_w)], idx_v)

        def start_g(j):
            b = j & 1
            return pltpu.async_copy(
                table_hbm.at[idx_v.at[pl.ds(j * chunk, chunk)]],
                bufs[b], sgs[b])

        def start_w(j):
            b = j & 1
            return pltpu.async_copy(
                bufs[b], out_hbm.at[pl.ds(base + j * chunk, chunk)], sws[b])

        g = [None] * n_chunks
        w = [None] * n_chunks
        g[0] = start_g(0)
        if n_chunks > 1:
            g[1] = start_g(1)
        for j in range(n_chunks):
            g[j].wait()
            w[j] = start_w(j)
            if j + 2 < n_chunks:
                w[j].wait()
                g[j + 2] = start_g(j + 2)
        for j in range(max(0, n_chunks - 2), n_chunks):
            w[j].wait()

    return body(table, idx)


def kernel(table, d_seq):
    n = table.shape[0] - 1
    d = table.shape[1]
    idx = jnp.minimum(jnp.arange(n, dtype=jnp.int32),
                      jnp.asarray(d_seq, jnp.int32) - 1)
    return _sc_gather(table, idx, n, d)


# speedup vs baseline: 1.5894x; 1.0264x over previous
import functools

import jax
import jax.numpy as jnp
from jax import lax
from jax.experimental import pallas as pl
from jax.experimental.pallas import tpu as pltpu
from jax.experimental.pallas import tpu_sc as plsc

NC = 2
NS = 16
NW = NC * NS


def _sc_gather(table, idx, n, d):
    b_per_w = n // NW
    chunk = 32
    n_chunks = b_per_w // chunk

    mesh = plsc.VectorSubcoreMesh(core_axis_name="c", subcore_axis_name="s")

    @functools.partial(
        pl.kernel,
        out_type=jax.ShapeDtypeStruct((n, d), jnp.float32),
        mesh=mesh,
        scratch_types=[
            pltpu.VMEM((b_per_w,), jnp.int32),
            pltpu.VMEM((chunk, d), jnp.float32),
            pltpu.VMEM((chunk, d), jnp.float32),
            pltpu.VMEM((chunk, d), jnp.float32),
            pltpu.SemaphoreType.DMA,
            pltpu.SemaphoreType.DMA,
            pltpu.SemaphoreType.DMA,
            pltpu.SemaphoreType.DMA,
            pltpu.SemaphoreType.DMA,
            pltpu.SemaphoreType.DMA,
        ],
    )
    def body(table_hbm, idx_hbm, out_hbm, idx_v, buf0, buf1, buf2,
             sg0, sg1, sg2, sw0, sw1, sw2):
        wid = lax.axis_index("s") * NC + lax.axis_index("c")
        base = wid * b_per_w
        nbuf = 3
        bufs, sgs, sws = (buf0, buf1, buf2), (sg0, sg1, sg2), (sw0, sw1, sw2)

        pltpu.sync_copy(idx_hbm.at[pl.ds(base, b_per_w)], idx_v)

        def start_g(j):
            b = j % nbuf
            return pltpu.async_copy(
                table_hbm.at[idx_v.at[pl.ds(j * chunk, chunk)]],
                bufs[b], sgs[b])

        def start_w(j):
            b = j % nbuf
            return pltpu.async_copy(
                bufs[b], out_hbm.at[pl.ds(base + j * chunk, chunk)], sws[b])

        g = [None] * n_chunks
        w = [None] * n_chunks
        for j in range(min(nbuf, n_chunks)):
            g[j] = start_g(j)
        for j in range(n_chunks):
            g[j].wait()
            w[j] = start_w(j)
            if j + nbuf < n_chunks:
                w[j].wait()
                g[j + nbuf] = start_g(j + nbuf)
        for j in range(max(0, n_chunks - nbuf), n_chunks):
            w[j].wait()

    return body(table, idx)


def kernel(table, d_seq):
    n = table.shape[0] - 1
    d = table.shape[1]
    idx = jnp.minimum(jnp.arange(n, dtype=jnp.int32),
                      jnp.asarray(d_seq, jnp.int32) - 1)
    return _sc_gather(table, idx, n, d)
